# Initial kernel scaffold; baseline (speedup 1.0000x reference)
import jax
import jax.numpy as jnp
from jax.experimental import pallas as pl


def kernel(fragments_batch):
    raise NotImplementedError("write your pallas kernel here")


# speedup vs baseline: 1.0434x; 1.0434x over previous
import jax
import jax.numpy as jnp
from jax import lax
from jax.experimental import pallas as pl
from jax.experimental.pallas import tpu as pltpu
from jax.experimental.pallas import tpu_sc as plsc

G = 20000
B = 64
N = 5000
F_IN = 16
F_OUT = F_IN + 1

NC = 2
NS = 16
L = 16

P = 320
GROUPS = P // L
CHUNK = 64
NCHUNK = P // CHUNK
GC = G // L
RPT = 1264
CRPT = 80
SPC = B // NC


def _encode_body(frag_hbm, out_hbm, accf, accc, frag_v, enc_v, hot_v,
                 idxf_v, idxc_v, zero_v, rb_v, cnt_v, o_v):
    c = lax.axis_index("c")
    s = lax.axis_index("s")
    base = jnp.minimum(s * P, N - P)
    skip = s * P - base
    obase = jnp.minimum(s * RPT, G - RPT)
    czbase = jnp.minimum(s * CRPT, GC - CRPT)
    fbase = c * G
    cbase = c * GC

    lanes = lax.iota(jnp.int32, L)
    zeros_f = jnp.zeros((L,), jnp.float32)
    col0 = jnp.zeros((L,), jnp.int32)

    def _z(i, carry):
        rows = jnp.minimum(i * L + lanes, RPT - 1)
        for col in range(L):
            plsc.store_scatter(zero_v, [rows, col0 + col], zeros_f)
        return carry

    lax.fori_loop(0, (RPT + L - 1) // L, _z, 0)

    def _sample(i, carry):
        b = i * NC + c

        pltpu.sync_copy(zero_v, accf.at[pl.ds(fbase + obase, RPT), :])
        pltpu.sync_copy(zero_v.at[pl.ds(0, CRPT), :],
                        accc.at[pl.ds(cbase + czbase, CRPT), :])
        plsc.subcore_barrier()

        pltpu.sync_copy(frag_hbm.at[b, pl.ds(base, P), :], frag_v)
        for g in range(GROUPS):
            rows = g * L + lanes
            srt = plsc.load_gather(frag_v, [rows, col0])
            end = plsc.load_gather(frag_v, [rows, col0 + 1])
            rel = (srt + end) * (0.5 * G)
            ci = rel.astype(jnp.int32)
            ci = jnp.minimum(jnp.maximum(ci, 0), G - 1)
            cell = ci.astype(jnp.float32)
            vm = jnp.where(rows >= skip, 1.0, 0.0)
            plsc.store_scatter(enc_v, [rows, col0], (rel - cell) * vm)
            plsc.store_scatter(enc_v, [rows, col0 + 1], (end - srt) * vm)
            for j in range(2, F_IN):
                fj = plsc.load_gather(frag_v, [rows, col0 + j])
                plsc.store_scatter(enc_v, [rows, col0 + j], fj * vm)
            for k in range(L):
                hot_v[g * L + k, :] = zeros_f
            plsc.store_scatter(hot_v, [rows, ci & (L - 1)], vm)
            idxf_v[g // 4, pl.ds((g % 4) * L, L)] = ci + fbase
            idxc_v[g // 4, pl.ds((g % 4) * L, L)] = (ci >> 4) + cbase

        for ch in range(NCHUNK):
            pltpu.sync_copy(
                enc_v.at[pl.ds(ch * CHUNK, CHUNK), :],
                accf.at[idxf_v.at[ch]],
                add=True,
            )
            pltpu.sync_copy(
                hot_v.at[pl.ds(ch * CHUNK, CHUNK), :],
                accc.at[idxc_v.at[ch]],
                add=True,
            )
        plsc.subcore_barrier()

        pltpu.sync_copy(accf.at[pl.ds(fbase + obase, RPT), :], rb_v)
        pltpu.sync_copy(accc.at[pl.ds(cbase + obase // L, RPT // L), :],
                        cnt_v)

        def _merge(g2, carry2):
            rows = g2 * L + lanes
            plsc.store_scatter(o_v, [rows, col0], cnt_v[g2, :])
            for j in range(F_IN):
                colv = plsc.load_gather(rb_v, [rows, col0 + j])
                plsc.store_scatter(o_v, [rows, col0 + (j + 1)], colv)
            return carry2

        lax.fori_loop(0, RPT // L, _merge, 0)
        pltpu.sync_copy(o_v, out_hbm.at[b, pl.ds(obase, RPT), :])
        plsc.subcore_barrier()
        return carry

    lax.fori_loop(0, SPC, _sample, 0)


@jax.jit
def kernel(fragments_batch):
    mesh = plsc.VectorSubcoreMesh(
        core_axis_name="c", subcore_axis_name="s",
        num_cores=NC, num_subcores=NS,
    )
    run = pl.kernel(
        _encode_body,
        out_type=jax.ShapeDtypeStruct((B, G, F_OUT), jnp.float32),
        mesh=mesh,
        compiler_params=pltpu.CompilerParams(
            needs_layout_passes=False, use_tc_tiling_on_sc=False,
        ),
        scratch_types=[
            pltpu.VMEM_SHARED((NC * G, L), jnp.float32),
            pltpu.VMEM_SHARED((NC * GC, L), jnp.float32),
            pltpu.VMEM((P, F_IN), jnp.float32),
            pltpu.VMEM((P, L), jnp.float32),
            pltpu.VMEM((P, L), jnp.float32),
            pltpu.VMEM((NCHUNK, CHUNK), jnp.int32),
            pltpu.VMEM((NCHUNK, CHUNK), jnp.int32),
            pltpu.VMEM((RPT, L), jnp.float32),
            pltpu.VMEM((RPT, L), jnp.float32),
            pltpu.VMEM((RPT // L, L), jnp.float32),
            pltpu.VMEM((RPT, F_OUT), jnp.float32),
        ],
    )
    return run(fragments_batch)
